# baseline (speedup 1.0000x reference)
import jax
import jax.numpy as jnp
from jax.experimental import pallas as pl
from jax.experimental.pallas import tpu as pltpu

_E = 320000


def _merge_body(gs, gt, gw, hs, ht, emb, out_s, out_t, out_w, out_e, ones_v, sem):
    lo = pl.ds(0, _E)
    hi = pl.ds(_E, _E)
    copies = [
        pltpu.make_async_copy(gs, out_s.at[lo], sem),
        pltpu.make_async_copy(hs, out_s.at[hi], sem),
        pltpu.make_async_copy(gt, out_t.at[lo], sem),
        pltpu.make_async_copy(ht, out_t.at[hi], sem),
        pltpu.make_async_copy(gw, out_w.at[lo], sem),
        pltpu.make_async_copy(emb, out_e, sem),
    ]
    for c in copies:
        c.start()
    ones_v[...] = jnp.ones((_E,), jnp.float32)
    last = pltpu.make_async_copy(ones_v, out_w.at[hi], sem)
    last.start()
    copies.append(last)
    for c in copies:
        c.wait()


def kernel(gen_sources, gen_targets, gen_weights, given_sources, given_targets, node_embeddings):
    any_spec = pl.BlockSpec(memory_space=pltpu.MemorySpace.HBM)
    out_s, out_t, out_w, out_e = pl.pallas_call(
        _merge_body,
        in_specs=[any_spec] * 6,
        out_specs=[any_spec] * 4,
        out_shape=(
            jax.ShapeDtypeStruct((2 * _E,), jnp.int32),
            jax.ShapeDtypeStruct((2 * _E,), jnp.int32),
            jax.ShapeDtypeStruct((2 * _E,), jnp.float32),
            jax.ShapeDtypeStruct(node_embeddings.shape, node_embeddings.dtype),
        ),
        scratch_shapes=[
            pltpu.VMEM((_E,), jnp.float32),
            pltpu.SemaphoreType.DMA,
        ],
    )(gen_sources, gen_targets, gen_weights, given_sources, given_targets, node_embeddings)
    return out_s, out_t, out_w, out_e


# speedup vs baseline: 14.7721x; 14.7721x over previous
import jax
import jax.numpy as jnp
from jax.experimental import pallas as pl
from jax.experimental.pallas import tpu as pltpu

_E = 320000
_D = 128
_N_NODES = 10000
_EMB = _N_NODES * _D
_NBUF = 5
_ONE_F32_BITS = 1065353216

_JOBS = (
    (0, 0, 0, 0),
    (3, 0, 0, _E),
    (1, 0, 1, 0),
    (4, 0, 1, _E),
    (2, 0, 2, 0),
    (5, 0 * _E, 3, 0 * _E),
    (5, 1 * _E, 3, 1 * _E),
    (5, 2 * _E, 3, 2 * _E),
    (5, 3 * _E, 3, 3 * _E),
)


def _merge_body(*refs):
    ins = refs[:6]
    outs = refs[6:10]
    bufs = refs[10:10 + _NBUF]
    ones_v = refs[10 + _NBUF]
    sem_in = refs[11 + _NBUF]
    sem_out = refs[12 + _NBUF]

    loads = []
    stores = []

    def start_load(k):
        i, so, _, _ = _JOBS[k]
        h = pltpu.make_async_copy(ins[i].at[pl.ds(so, _E)], bufs[k % _NBUF], sem_in.at[k % _NBUF])
        h.start()
        loads.append(h)

    for k in range(_NBUF):
        start_load(k)

    ones_v[...] = jnp.full((_E,), _ONE_F32_BITS, jnp.int32)
    ones_store = pltpu.make_async_copy(ones_v, outs[2].at[pl.ds(_E, _E)], sem_out.at[_NBUF])
    ones_store.start()

    for k in range(len(_JOBS)):
        loads[k].wait()
        _, _, o, do = _JOBS[k]
        h = pltpu.make_async_copy(bufs[k % _NBUF], outs[o].at[pl.ds(do, _E)], sem_out.at[k % _NBUF])
        h.start()
        stores.append(h)
        nxt = k + _NBUF
        if nxt < len(_JOBS):
            stores[k].wait()
            start_load(nxt)
    for k in range(len(_JOBS) - _NBUF, len(_JOBS)):
        stores[k].wait()
    ones_store.wait()


def kernel(gen_sources, gen_targets, gen_weights, given_sources, given_targets, node_embeddings):
    hbm = pl.BlockSpec(memory_space=pltpu.MemorySpace.HBM)
    gw_bits = jax.lax.bitcast_convert_type(gen_weights, jnp.int32)
    emb_bits = jax.lax.bitcast_convert_type(node_embeddings, jnp.int32).reshape(_EMB)
    out_s, out_t, out_w, out_e = pl.pallas_call(
        _merge_body,
        in_specs=[hbm] * 6,
        out_specs=[hbm] * 4,
        out_shape=(
            jax.ShapeDtypeStruct((2 * _E,), jnp.int32),
            jax.ShapeDtypeStruct((2 * _E,), jnp.int32),
            jax.ShapeDtypeStruct((2 * _E,), jnp.int32),
            jax.ShapeDtypeStruct((_EMB,), jnp.int32),
        ),
        scratch_shapes=[pltpu.VMEM((_E,), jnp.int32)] * (_NBUF + 1)
        + [
            pltpu.SemaphoreType.DMA((_NBUF,)),
            pltpu.SemaphoreType.DMA((_NBUF + 1,)),
        ],
    )(gen_sources, gen_targets, gw_bits, given_sources, given_targets, emb_bits)
    return (
        out_s,
        out_t,
        jax.lax.bitcast_convert_type(out_w, jnp.float32),
        jax.lax.bitcast_convert_type(out_e.reshape(_N_NODES, _D), jnp.float32),
    )


# speedup vs baseline: 15.4727x; 1.0474x over previous
import jax
import jax.numpy as jnp
from jax.experimental import pallas as pl
from jax.experimental.pallas import tpu as pltpu

_E = 320000
_D = 128
_N_NODES = 10000
_EMB = _N_NODES * _D
_NBUF = 9
_ONE_F32_BITS = 1065353216

_JOBS = (
    (0, 0, 0, 0),
    (3, 0, 0, _E),
    (1, 0, 1, 0),
    (4, 0, 1, _E),
    (2, 0, 2, 0),
    (5, 0 * _E, 3, 0 * _E),
    (5, 1 * _E, 3, 1 * _E),
    (5, 2 * _E, 3, 2 * _E),
    (5, 3 * _E, 3, 3 * _E),
)


def _merge_body(*refs):
    ins = refs[:6]
    outs = refs[6:10]
    bufs = refs[10:10 + _NBUF]
    ones_v = refs[10 + _NBUF]
    sem_in = refs[11 + _NBUF]
    sem_out = refs[12 + _NBUF]

    loads = []
    stores = []

    def start_load(k):
        i, so, _, _ = _JOBS[k]
        h = pltpu.make_async_copy(ins[i].at[pl.ds(so, _E)], bufs[k % _NBUF], sem_in.at[k % _NBUF])
        h.start()
        loads.append(h)

    for k in range(_NBUF):
        start_load(k)

    ones_v[...] = jnp.full((_E,), _ONE_F32_BITS, jnp.int32)
    ones_store = pltpu.make_async_copy(ones_v, outs[2].at[pl.ds(_E, _E)], sem_out.at[_NBUF])
    ones_store.start()

    for k in range(len(_JOBS)):
        loads[k].wait()
        _, _, o, do = _JOBS[k]
        h = pltpu.make_async_copy(bufs[k % _NBUF], outs[o].at[pl.ds(do, _E)], sem_out.at[k % _NBUF])
        h.start()
        stores.append(h)
        nxt = k + _NBUF
        if nxt < len(_JOBS):
            stores[k].wait()
            start_load(nxt)
    for k in range(len(_JOBS) - _NBUF, len(_JOBS)):
        stores[k].wait()
    ones_store.wait()


def kernel(gen_sources, gen_targets, gen_weights, given_sources, given_targets, node_embeddings):
    hbm = pl.BlockSpec(memory_space=pltpu.MemorySpace.HBM)
    gw_bits = jax.lax.bitcast_convert_type(gen_weights, jnp.int32)
    emb_bits = jax.lax.bitcast_convert_type(node_embeddings, jnp.int32).reshape(_EMB)
    out_s, out_t, out_w, out_e = pl.pallas_call(
        _merge_body,
        in_specs=[hbm] * 6,
        out_specs=[hbm] * 4,
        out_shape=(
            jax.ShapeDtypeStruct((2 * _E,), jnp.int32),
            jax.ShapeDtypeStruct((2 * _E,), jnp.int32),
            jax.ShapeDtypeStruct((2 * _E,), jnp.int32),
            jax.ShapeDtypeStruct((_EMB,), jnp.int32),
        ),
        scratch_shapes=[pltpu.VMEM((_E,), jnp.int32)] * (_NBUF + 1)
        + [
            pltpu.SemaphoreType.DMA((_NBUF,)),
            pltpu.SemaphoreType.DMA((_NBUF + 1,)),
        ],
    )(gen_sources, gen_targets, gw_bits, given_sources, given_targets, emb_bits)
    return (
        out_s,
        out_t,
        jax.lax.bitcast_convert_type(out_w, jnp.float32),
        jax.lax.bitcast_convert_type(out_e.reshape(_N_NODES, _D), jnp.float32),
    )
